# Initial kernel scaffold; baseline (speedup 1.0000x reference)
import jax
import jax.numpy as jnp
from jax.experimental import pallas as pl


def kernel(feature, label):
    raise NotImplementedError("write your pallas kernel here")


# speedup vs baseline: 11.6277x; 11.6277x over previous
import numpy as np
import jax
import jax.numpy as jnp
from jax import lax
from jax.experimental import pallas as pl
from jax.experimental.pallas import tpu as pltpu

_NUM_CLASSES = 19
_D = 256
_N = 1024
_BINS = 51
_BINS_VALS = np.linspace(-5.0, 5.0, _BINS).astype(np.float32)
_TWO_PI = 6.283185307179586


def _dot(a, b, precision):
    return lax.dot_general(
        a, b, (((1,), (0,)), ((), ())),
        precision=precision, preferred_element_type=jnp.float32)


def _tc_body(featT_ref, lblrow_ref, lblcol_ref, bins_ref, out_ref,
             sample_scr, target_scr):
    featT = featT_ref[...]
    lbl_row = lblrow_ref[...]
    lbl_col = lblcol_ref[...]

    onehot = (lax.broadcasted_iota(jnp.int32, (_NUM_CLASSES, _N), 0)
              == lbl_row).astype(jnp.float32)
    onehotT = (lax.broadcasted_iota(jnp.int32, (_N, _NUM_CLASSES), 1)
               == lbl_col).astype(jnp.float32)

    cnt = jnp.sum(onehot, axis=1, keepdims=True)
    inv_n = 1.0 / jnp.maximum(cnt, 1.0)

    hi = lax.Precision.HIGHEST
    s1 = _dot(onehot, featT, hi)
    s2 = _dot(onehot, featT * featT, hi)
    miu = s1 * inv_n
    var = jnp.maximum(s2 * inv_n - miu * miu, 1e-12)
    vs = var * (1.0 / 25.0)
    coef_s = -0.5 / vs
    nrm_s = lax.rsqrt(_TWO_PI * vs)
    coef_t = -0.5 / var
    nrm_t = lax.rsqrt(_TWO_PI * var)

    coef_pix = _dot(onehotT, coef_s, hi)
    nrm_pix = _dot(onehotT, nrm_s, hi)

    onehot_bf = onehot.astype(jnp.bfloat16)

    def bin_step(b, carry):
        bv = bins_ref[b]
        d = bv - featT
        kern = jnp.exp(d * d * coef_pix) * nrm_pix
        k_hi = kern.astype(jnp.bfloat16)
        k_lo = (kern - k_hi.astype(jnp.float32)).astype(jnp.bfloat16)
        sam = (_dot(onehot_bf, k_hi, lax.Precision.DEFAULT)
               + _dot(onehot_bf, k_lo, lax.Precision.DEFAULT))
        sample_scr[pl.ds(b, 1)] = sam[None]
        dt = bv - miu
        tgt = jnp.exp(dt * dt * coef_t) * nrm_t
        target_scr[pl.ds(b, 1)] = tgt[None]
        return carry

    lax.fori_loop(0, _BINS, bin_step, 0)

    sample = sample_scr[...]
    target = target_scr[...]
    inv_zs = 1.0 / jnp.maximum(jnp.sum(sample, axis=0), 1e-20)
    inv_zt = 1.0 / jnp.maximum(jnp.sum(target, axis=0), 1e-20)
    dd = sample * inv_zs[None] - target * inv_zt[None]
    sl1 = jnp.where(jnp.abs(dd) < 1.0, 0.5 * dd * dd, jnp.abs(dd) - 0.5)
    per_c = jnp.sum(jnp.sum(sl1, axis=0), axis=1, keepdims=True)
    cls = lax.broadcasted_iota(jnp.int32, (_NUM_CLASSES, 1), 0)
    gated = jnp.where((cnt > 0.0) & (cls > 0), per_c, 0.0)
    out_ref[0, 0] = jnp.sum(gated) * (1.0 / (_D * _BINS))


def kernel(feature, label):
    B, D, H, W = feature.shape
    sh = label.shape[2] // H
    sw = label.shape[3] // W
    lbl = label[0, 0, ::sh, ::sw].reshape(-1).astype(jnp.int32)
    featT = feature[0].reshape(D, -1).T
    n = featT.shape[0]

    out = pl.pallas_call(
        _tc_body,
        out_shape=jax.ShapeDtypeStruct((1, 1), jnp.float32),
        out_specs=pl.BlockSpec(memory_space=pltpu.MemorySpace.SMEM),
        in_specs=[
            pl.BlockSpec(memory_space=pltpu.MemorySpace.VMEM),
            pl.BlockSpec(memory_space=pltpu.MemorySpace.VMEM),
            pl.BlockSpec(memory_space=pltpu.MemorySpace.VMEM),
            pl.BlockSpec(memory_space=pltpu.MemorySpace.SMEM),
        ],
        scratch_shapes=[
            pltpu.VMEM((_BINS, _NUM_CLASSES, _D), jnp.float32),
            pltpu.VMEM((_BINS, _NUM_CLASSES, _D), jnp.float32),
        ],
    )(featT, lbl.reshape(1, n), lbl.reshape(n, 1), jnp.asarray(_BINS_VALS))
    return out[0, 0]


# speedup vs baseline: 16.2300x; 1.3958x over previous
import numpy as np
import jax
import jax.numpy as jnp
from jax import lax
from jax.experimental import pallas as pl
from jax.experimental.pallas import tpu as pltpu

_NUM_CLASSES = 19
_D = 256
_N = 1024
_BINS = 51
_BINS_VALS = np.linspace(-5.0, 5.0, _BINS).astype(np.float32)
_TWO_PI = 6.283185307179586


def _dot(a, b, precision):
    return lax.dot_general(
        a, b, (((1,), (0,)), ((), ())),
        precision=precision, preferred_element_type=jnp.float32)


def _tc_body(featT_ref, lblrow_ref, lblcol_ref, bins_ref, out_ref,
             sample_scr, target_scr):
    featT = featT_ref[...]
    lbl_row = lblrow_ref[...]
    lbl_col = lblcol_ref[...]

    onehot = (lax.broadcasted_iota(jnp.int32, (_NUM_CLASSES, _N), 0)
              == lbl_row).astype(jnp.float32)
    onehotT = (lax.broadcasted_iota(jnp.int32, (_N, _NUM_CLASSES), 1)
               == lbl_col).astype(jnp.float32)

    cnt = jnp.sum(onehot, axis=1, keepdims=True)
    inv_n = 1.0 / jnp.maximum(cnt, 1.0)

    hi = lax.Precision.HIGHEST
    s1 = _dot(onehot, featT, hi)
    s2 = _dot(onehot, featT * featT, hi)
    miu = s1 * inv_n
    var = jnp.maximum(s2 * inv_n - miu * miu, 1e-12)
    vs = var * (1.0 / 25.0)
    coef_s = -0.5 / vs
    nrm_s = lax.rsqrt(_TWO_PI * vs)
    coef_t = -0.5 / var
    nrm_t = lax.rsqrt(_TWO_PI * var)

    coef_pix = _dot(onehotT, coef_s, hi)
    nrm_pix = _dot(onehotT, nrm_s, hi)

    onehot_bf = onehot.astype(jnp.bfloat16)

    def bin_group(g, carry):
        for u in range(3):
            b = g * jnp.int32(3) + jnp.int32(u)
            bv = bins_ref[b]
            d = bv - featT
            kern = jnp.exp(d * d * coef_pix) * nrm_pix
            sam = _dot(onehot_bf, kern.astype(jnp.bfloat16),
                       lax.Precision.DEFAULT)
            sample_scr[pl.ds(b, 1)] = sam[None]
            dt = bv - miu
            tgt = jnp.exp(dt * dt * coef_t) * nrm_t
            target_scr[pl.ds(b, 1)] = tgt[None]
        return carry

    lax.fori_loop(jnp.int32(0), jnp.int32(_BINS // 3), bin_group, 0)

    sample = sample_scr[...]
    target = target_scr[...]
    inv_zs = 1.0 / jnp.maximum(jnp.sum(sample, axis=0), 1e-20)
    inv_zt = 1.0 / jnp.maximum(jnp.sum(target, axis=0), 1e-20)
    dd = sample * inv_zs[None] - target * inv_zt[None]
    sl1 = jnp.where(jnp.abs(dd) < 1.0, 0.5 * dd * dd, jnp.abs(dd) - 0.5)
    per_c = jnp.sum(jnp.sum(sl1, axis=0), axis=1, keepdims=True)
    cls = lax.broadcasted_iota(jnp.int32, (_NUM_CLASSES, 1), 0)
    gated = jnp.where((cnt > 0.0) & (cls > 0), per_c, 0.0)
    out_ref[0, 0] = jnp.sum(gated) * (1.0 / (_D * _BINS))


def kernel(feature, label):
    B, D, H, W = feature.shape
    sh = label.shape[2] // H
    sw = label.shape[3] // W
    lbl = label[0, 0, ::sh, ::sw].reshape(-1).astype(jnp.int32)
    featT = feature[0].reshape(D, -1).T
    n = featT.shape[0]

    out = pl.pallas_call(
        _tc_body,
        out_shape=jax.ShapeDtypeStruct((1, 1), jnp.float32),
        out_specs=pl.BlockSpec(memory_space=pltpu.MemorySpace.SMEM),
        in_specs=[
            pl.BlockSpec(memory_space=pltpu.MemorySpace.VMEM),
            pl.BlockSpec(memory_space=pltpu.MemorySpace.VMEM),
            pl.BlockSpec(memory_space=pltpu.MemorySpace.VMEM),
            pl.BlockSpec(memory_space=pltpu.MemorySpace.SMEM),
        ],
        scratch_shapes=[
            pltpu.VMEM((_BINS, _NUM_CLASSES, _D), jnp.float32),
            pltpu.VMEM((_BINS, _NUM_CLASSES, _D), jnp.float32),
        ],
    )(featT, lbl.reshape(1, n), lbl.reshape(n, 1), jnp.asarray(_BINS_VALS))
    return out[0, 0]
